# baseline (speedup 1.0000x reference)
import jax
import jax.numpy as jnp
from jax import lax
from jax.experimental import pallas as pl
from jax.experimental.pallas import tpu as pltpu
from jax.experimental.pallas import tpu_sc as plsc

_NB = 255
_N = 131072
_NW = 32
_RPW = _N // _NW
_CR = 128
_NCH = _RPW // _CR
_VPC = _CR // 16
_RING = 3


def _search16(bins_vmem, x):
    idx = jnp.zeros((16,), jnp.int32)
    for s in (128, 64, 32, 16, 8, 4, 2, 1):
        t = idx + s
        bv = plsc.load_gather(bins_vmem, [t - 1])
        idx = jnp.where(bv < x, t, idx)
    return idx


def _sc_body(values_hbm, bins_hbm, zrows_hbm, out_hbm,
             x_vmem, bins_vmem, buf0, buf1, buf2, hist0, hist1, hist2,
             sem0, sem1, sem2, semx, semb):
    wid = lax.axis_index("s") * 2 + lax.axis_index("c")
    base_row = wid * _RPW
    bufs = (buf0, buf1, buf2)
    hists = (hist0, hist1, hist2)
    sems = (sem0, sem1, sem2)

    cx = pltpu.make_async_copy(values_hbm.at[pl.ds(base_row, _RPW)], x_vmem, semx)
    cb = pltpu.make_async_copy(bins_hbm, bins_vmem, semb)
    cz = [pltpu.make_async_copy(zrows_hbm, bufs[b], sems[b]) for b in range(_RING)]
    cx.start()
    cb.start()
    for c in cz:
        c.start()

    lane = lax.iota(jnp.int32, 16)
    zero16 = jnp.zeros((16,), jnp.float32)

    def out_slice(c):
        return out_hbm.at[pl.ds(base_row + c * _CR, _CR), :]

    def fill_chunk(c, buf, hist):
        for v in range(_VPC):
            x = x_vmem[pl.ds(c * _CR + v * 16, 16)]
            idx = _search16(bins_vmem, x)
            l = jnp.minimum(jnp.maximum(idx - 1, 0), _NB - 2)
            bl = plsc.load_gather(bins_vmem, [l])
            br = plsc.load_gather(bins_vmem, [l + 1])
            wl = (br - x) / (br - bl)
            wl = jnp.minimum(jnp.maximum(wl, 0.0), 1.0)
            row = lane + (v * 16)
            plsc.store_scatter(buf, [row, l], wl)
            plsc.store_scatter(buf, [row, l + 1], 1.0 - wl)
            hist[pl.ds(v * 16, 16)] = l

    def zero_chunk(buf, hist):
        for v in range(_VPC):
            l = hist[pl.ds(v * 16, 16)]
            row = lane + (v * 16)
            plsc.store_scatter(buf, [row, l], zero16)
            plsc.store_scatter(buf, [row, l + 1], zero16)

    cx.wait()
    cb.wait()
    for b in range(_RING):
        cz[b].wait()
        fill_chunk(b, bufs[b], hists[b])
        pltpu.make_async_copy(bufs[b], out_slice(b), sems[b]).start()

    def step(i, carry):
        for b in range(_RING):
            c = i * _RING + b
            pltpu.make_async_copy(bufs[b], out_slice(c - _RING), sems[b]).wait()
            zero_chunk(bufs[b], hists[b])
            fill_chunk(c, bufs[b], hists[b])
            pltpu.make_async_copy(bufs[b], out_slice(c), sems[b]).start()
        return carry

    n_full = _NCH // _RING
    lax.fori_loop(1, n_full, step, 0)

    for c in range(n_full * _RING, _NCH):
        b = c % _RING
        pltpu.make_async_copy(bufs[b], out_slice(c - _RING), sems[b]).wait()
        zero_chunk(bufs[b], hists[b])
        fill_chunk(c, bufs[b], hists[b])
        pltpu.make_async_copy(bufs[b], out_slice(c), sems[b]).start()

    for c in range(_NCH - _RING, _NCH):
        b = c % _RING
        pltpu.make_async_copy(bufs[b], out_slice(c), sems[b]).wait()


@jax.jit
def _sc_twohot(flat, bins, zrows):
    mesh = plsc.VectorSubcoreMesh(core_axis_name="c", subcore_axis_name="s")
    run = pl.kernel(
        _sc_body,
        out_type=jax.ShapeDtypeStruct((_N, _NB), jnp.float32),
        mesh=mesh,
        scratch_types=[
            pltpu.VMEM((_RPW,), jnp.float32),
            pltpu.VMEM((_NB,), jnp.float32),
            pltpu.VMEM((_CR, _NB), jnp.float32),
            pltpu.VMEM((_CR, _NB), jnp.float32),
            pltpu.VMEM((_CR, _NB), jnp.float32),
            pltpu.VMEM((_CR,), jnp.int32),
            pltpu.VMEM((_CR,), jnp.int32),
            pltpu.VMEM((_CR,), jnp.int32),
            pltpu.SemaphoreType.DMA,
            pltpu.SemaphoreType.DMA,
            pltpu.SemaphoreType.DMA,
            pltpu.SemaphoreType.DMA,
            pltpu.SemaphoreType.DMA,
        ],
        compiler_params=pltpu.CompilerParams(needs_layout_passes=False),
    )
    return run(flat, bins, zrows)


def kernel(values, bin_values):
    orig_shape = values.shape
    flat = values.reshape(-1)
    zrows = jnp.zeros((_CR, _NB), jnp.float32)
    out = _sc_twohot(flat, bin_values, zrows)
    return out.reshape(orig_shape + (_NB,))


# speedup vs baseline: 1.1622x; 1.1622x over previous
import jax
import jax.numpy as jnp
from jax import lax
from jax.experimental import pallas as pl
from jax.experimental.pallas import tpu as pltpu
from jax.experimental.pallas import tpu_sc as plsc

_NB = 255
_N = 131072
_NW = 32
_RPW = _N // _NW
_CR = 128
_NCH = _RPW // _CR
_VPC = _CR // 16


def _search16(bins_vmem, x):
    idx = jnp.zeros((16,), jnp.int32)
    for s in (128, 64, 32, 16, 8, 4, 2, 1):
        t = idx + s
        bv = plsc.load_gather(bins_vmem, [t - 1])
        idx = jnp.where(bv < x, t, idx)
    return idx


def _sc_body(values_hbm, bins_hbm, zrows_hbm, out_hbm,
             x_vmem, bins_vmem, buf0, buf1, hist0, hist1,
             sem0, sem1, semx, semb):
    wid = lax.axis_index("s") * 2 + lax.axis_index("c")
    base_row = wid * _RPW
    cx = pltpu.make_async_copy(values_hbm.at[pl.ds(base_row, _RPW)], x_vmem, semx)
    cb = pltpu.make_async_copy(bins_hbm, bins_vmem, semb)
    cz0 = pltpu.make_async_copy(zrows_hbm, buf0, sem0)
    cz1 = pltpu.make_async_copy(zrows_hbm, buf1, sem1)
    cx.start()
    cb.start()
    cz0.start()
    cz1.start()

    bufs = (buf0, buf1)
    hists = (hist0, hist1)
    sems = (sem0, sem1)
    lane = lax.iota(jnp.int32, 16)
    zero16 = jnp.zeros((16,), jnp.float32)

    def out_slice(c):
        return out_hbm.at[pl.ds(base_row + c * _CR, _CR), :]

    def fill_chunk(c, buf, hist):
        for v in range(_VPC):
            x = x_vmem[pl.ds(c * _CR + v * 16, 16)]
            idx = _search16(bins_vmem, x)
            l = jnp.minimum(jnp.maximum(idx - 1, 0), _NB - 2)
            bl = plsc.load_gather(bins_vmem, [l])
            br = plsc.load_gather(bins_vmem, [l + 1])
            wl = (br - x) / (br - bl)
            wl = jnp.minimum(jnp.maximum(wl, 0.0), 1.0)
            row = lane + (v * 16)
            plsc.store_scatter(buf, [row, l], wl)
            plsc.store_scatter(buf, [row, l + 1], 1.0 - wl)
            hist[pl.ds(v * 16, 16)] = l

    def zero_chunk(buf, hist):
        for v in range(_VPC):
            l = hist[pl.ds(v * 16, 16)]
            row = lane + (v * 16)
            plsc.store_scatter(buf, [row, l], zero16)
            plsc.store_scatter(buf, [row, l + 1], zero16)

    cx.wait()
    cb.wait()
    for b, cz in ((0, cz0), (1, cz1)):
        cz.wait()
        fill_chunk(b, bufs[b], hists[b])
        pltpu.make_async_copy(bufs[b], out_slice(b), sems[b]).start()

    def step(i, carry):
        for b in range(2):
            c = i * 2 + b
            pltpu.make_async_copy(bufs[b], out_slice(c - 2), sems[b]).wait()
            zero_chunk(bufs[b], hists[b])
            fill_chunk(c, bufs[b], hists[b])
            pltpu.make_async_copy(bufs[b], out_slice(c), sems[b]).start()
        return carry

    lax.fori_loop(1, _NCH // 2, step, 0)

    for b in range(2):
        pltpu.make_async_copy(bufs[b], out_slice(_NCH - 2 + b), sems[b]).wait()


@jax.jit
def _sc_twohot(flat, bins, zrows):
    mesh = plsc.VectorSubcoreMesh(core_axis_name="c", subcore_axis_name="s")
    run = pl.kernel(
        _sc_body,
        out_type=jax.ShapeDtypeStruct((_N, _NB), jnp.float32),
        mesh=mesh,
        scratch_types=[
            pltpu.VMEM((_RPW,), jnp.float32),
            pltpu.VMEM((_NB,), jnp.float32),
            pltpu.VMEM((_CR, _NB), jnp.float32),
            pltpu.VMEM((_CR, _NB), jnp.float32),
            pltpu.VMEM((_CR,), jnp.int32),
            pltpu.VMEM((_CR,), jnp.int32),
            pltpu.SemaphoreType.DMA,
            pltpu.SemaphoreType.DMA,
            pltpu.SemaphoreType.DMA,
            pltpu.SemaphoreType.DMA,
        ],
        compiler_params=pltpu.CompilerParams(needs_layout_passes=False),
    )
    return run(flat, bins, zrows)


def kernel(values, bin_values):
    orig_shape = values.shape
    flat = values.reshape(-1)
    zrows = jnp.zeros((_CR, _NB), jnp.float32)
    out = _sc_twohot(flat, bin_values, zrows)
    return out.reshape(orig_shape + (_NB,))


# speedup vs baseline: 1.2578x; 1.0823x over previous
import jax
import jax.numpy as jnp
from jax import lax
from jax.experimental import pallas as pl
from jax.experimental.pallas import tpu as pltpu
from jax.experimental.pallas import tpu_sc as plsc

_NB = 255
_N = 131072
_NW = 32
_RPW = _N // _NW
_CR = 64
_NCH = _RPW // _CR
_VPC = _CR // 16


def _search16(bins_vmem, x):
    idx = jnp.zeros((16,), jnp.int32)
    for s in (128, 64, 32, 16, 8, 4, 2, 1):
        t = idx + s
        bv = plsc.load_gather(bins_vmem, [t - 1])
        idx = jnp.where(bv < x, t, idx)
    return idx


def _sc_body(values_hbm, bins_hbm, zrows_hbm, out_hbm,
             x_vmem, bins_vmem, buf0, buf1, hist0, hist1,
             sem0, sem1, semx, semb):
    wid = lax.axis_index("s") * 2 + lax.axis_index("c")
    base_row = wid * _RPW
    cx = pltpu.make_async_copy(values_hbm.at[pl.ds(base_row, _RPW)], x_vmem, semx)
    cb = pltpu.make_async_copy(bins_hbm, bins_vmem, semb)
    cz0 = pltpu.make_async_copy(zrows_hbm, buf0, sem0)
    cz1 = pltpu.make_async_copy(zrows_hbm, buf1, sem1)
    cx.start()
    cb.start()
    cz0.start()
    cz1.start()

    bufs = (buf0, buf1)
    hists = (hist0, hist1)
    sems = (sem0, sem1)
    lane = lax.iota(jnp.int32, 16)
    zero16 = jnp.zeros((16,), jnp.float32)

    def out_slice(c):
        return out_hbm.at[pl.ds(base_row + c * _CR, _CR), :]

    def fill_chunk(c, buf, hist):
        for v in range(_VPC):
            x = x_vmem[pl.ds(c * _CR + v * 16, 16)]
            idx = _search16(bins_vmem, x)
            l = jnp.minimum(jnp.maximum(idx - 1, 0), _NB - 2)
            bl = plsc.load_gather(bins_vmem, [l])
            br = plsc.load_gather(bins_vmem, [l + 1])
            wl = (br - x) / (br - bl)
            wl = jnp.minimum(jnp.maximum(wl, 0.0), 1.0)
            row = lane + (v * 16)
            plsc.store_scatter(buf, [row, l], wl)
            plsc.store_scatter(buf, [row, l + 1], 1.0 - wl)
            hist[pl.ds(v * 16, 16)] = l

    def zero_chunk(buf, hist):
        for v in range(_VPC):
            l = hist[pl.ds(v * 16, 16)]
            row = lane + (v * 16)
            plsc.store_scatter(buf, [row, l], zero16)
            plsc.store_scatter(buf, [row, l + 1], zero16)

    cx.wait()
    cb.wait()
    for b, cz in ((0, cz0), (1, cz1)):
        cz.wait()
        fill_chunk(b, bufs[b], hists[b])
        pltpu.make_async_copy(bufs[b], out_slice(b), sems[b]).start()

    def step(i, carry):
        for b in range(2):
            c = i * 2 + b
            pltpu.make_async_copy(bufs[b], out_slice(c - 2), sems[b]).wait()
            zero_chunk(bufs[b], hists[b])
            fill_chunk(c, bufs[b], hists[b])
            pltpu.make_async_copy(bufs[b], out_slice(c), sems[b]).start()
        return carry

    lax.fori_loop(1, _NCH // 2, step, 0)

    for b in range(2):
        pltpu.make_async_copy(bufs[b], out_slice(_NCH - 2 + b), sems[b]).wait()


@jax.jit
def _sc_twohot(flat, bins, zrows):
    mesh = plsc.VectorSubcoreMesh(core_axis_name="c", subcore_axis_name="s")
    run = pl.kernel(
        _sc_body,
        out_type=jax.ShapeDtypeStruct((_N, _NB), jnp.float32),
        mesh=mesh,
        scratch_types=[
            pltpu.VMEM((_RPW,), jnp.float32),
            pltpu.VMEM((_NB,), jnp.float32),
            pltpu.VMEM((_CR, _NB), jnp.float32),
            pltpu.VMEM((_CR, _NB), jnp.float32),
            pltpu.VMEM((_CR,), jnp.int32),
            pltpu.VMEM((_CR,), jnp.int32),
            pltpu.SemaphoreType.DMA,
            pltpu.SemaphoreType.DMA,
            pltpu.SemaphoreType.DMA,
            pltpu.SemaphoreType.DMA,
        ],
        compiler_params=pltpu.CompilerParams(needs_layout_passes=False),
    )
    return run(flat, bins, zrows)


def kernel(values, bin_values):
    orig_shape = values.shape
    flat = values.reshape(-1)
    zrows = jnp.zeros((_CR, _NB), jnp.float32)
    out = _sc_twohot(flat, bin_values, zrows)
    return out.reshape(orig_shape + (_NB,))


# speedup vs baseline: 1.3299x; 1.0573x over previous
import jax
import jax.numpy as jnp
from jax import lax
from jax.experimental import pallas as pl
from jax.experimental.pallas import tpu as pltpu
from jax.experimental.pallas import tpu_sc as plsc

_NB = 255
_N = 131072
_NW = 32
_RPW = _N // _NW
_CR = 32
_NCH = _RPW // _CR
_VPC = _CR // 16


def _search16(bins_vmem, x):
    idx = jnp.zeros((16,), jnp.int32)
    for s in (128, 64, 32, 16, 8, 4, 2, 1):
        t = idx + s
        bv = plsc.load_gather(bins_vmem, [t - 1])
        idx = jnp.where(bv < x, t, idx)
    return idx


def _sc_body(values_hbm, bins_hbm, zrows_hbm, out_hbm,
             x_vmem, bins_vmem, buf0, buf1, hist0, hist1,
             sem0, sem1, semx, semb):
    wid = lax.axis_index("s") * 2 + lax.axis_index("c")
    base_row = wid * _RPW
    cx = pltpu.make_async_copy(values_hbm.at[pl.ds(base_row, _RPW)], x_vmem, semx)
    cb = pltpu.make_async_copy(bins_hbm, bins_vmem, semb)
    cz0 = pltpu.make_async_copy(zrows_hbm, buf0, sem0)
    cz1 = pltpu.make_async_copy(zrows_hbm, buf1, sem1)
    cx.start()
    cb.start()
    cz0.start()
    cz1.start()

    bufs = (buf0, buf1)
    hists = (hist0, hist1)
    sems = (sem0, sem1)
    lane = lax.iota(jnp.int32, 16)
    zero16 = jnp.zeros((16,), jnp.float32)

    def out_slice(c):
        return out_hbm.at[pl.ds(base_row + c * _CR, _CR), :]

    def fill_chunk(c, buf, hist):
        for v in range(_VPC):
            x = x_vmem[pl.ds(c * _CR + v * 16, 16)]
            idx = _search16(bins_vmem, x)
            l = jnp.minimum(jnp.maximum(idx - 1, 0), _NB - 2)
            bl = plsc.load_gather(bins_vmem, [l])
            br = plsc.load_gather(bins_vmem, [l + 1])
            wl = (br - x) / (br - bl)
            wl = jnp.minimum(jnp.maximum(wl, 0.0), 1.0)
            row = lane + (v * 16)
            plsc.store_scatter(buf, [row, l], wl)
            plsc.store_scatter(buf, [row, l + 1], 1.0 - wl)
            hist[pl.ds(v * 16, 16)] = l

    def zero_chunk(buf, hist):
        for v in range(_VPC):
            l = hist[pl.ds(v * 16, 16)]
            row = lane + (v * 16)
            plsc.store_scatter(buf, [row, l], zero16)
            plsc.store_scatter(buf, [row, l + 1], zero16)

    cx.wait()
    cb.wait()
    for b, cz in ((0, cz0), (1, cz1)):
        cz.wait()
        fill_chunk(b, bufs[b], hists[b])
        pltpu.make_async_copy(bufs[b], out_slice(b), sems[b]).start()

    def step(i, carry):
        for b in range(2):
            c = i * 2 + b
            pltpu.make_async_copy(bufs[b], out_slice(c - 2), sems[b]).wait()
            zero_chunk(bufs[b], hists[b])
            fill_chunk(c, bufs[b], hists[b])
            pltpu.make_async_copy(bufs[b], out_slice(c), sems[b]).start()
        return carry

    lax.fori_loop(1, _NCH // 2, step, 0)

    for b in range(2):
        pltpu.make_async_copy(bufs[b], out_slice(_NCH - 2 + b), sems[b]).wait()


@jax.jit
def _sc_twohot(flat, bins, zrows):
    mesh = plsc.VectorSubcoreMesh(core_axis_name="c", subcore_axis_name="s")
    run = pl.kernel(
        _sc_body,
        out_type=jax.ShapeDtypeStruct((_N, _NB), jnp.float32),
        mesh=mesh,
        scratch_types=[
            pltpu.VMEM((_RPW,), jnp.float32),
            pltpu.VMEM((_NB,), jnp.float32),
            pltpu.VMEM((_CR, _NB), jnp.float32),
            pltpu.VMEM((_CR, _NB), jnp.float32),
            pltpu.VMEM((_CR,), jnp.int32),
            pltpu.VMEM((_CR,), jnp.int32),
            pltpu.SemaphoreType.DMA,
            pltpu.SemaphoreType.DMA,
            pltpu.SemaphoreType.DMA,
            pltpu.SemaphoreType.DMA,
        ],
        compiler_params=pltpu.CompilerParams(needs_layout_passes=False),
    )
    return run(flat, bins, zrows)


def kernel(values, bin_values):
    orig_shape = values.shape
    flat = values.reshape(-1)
    zrows = jnp.zeros((_CR, _NB), jnp.float32)
    out = _sc_twohot(flat, bin_values, zrows)
    return out.reshape(orig_shape + (_NB,))
